# baseline (speedup 1.0000x reference)
import functools

import jax
import jax.numpy as jnp
from jax import lax
from jax.experimental import pallas as pl
from jax.experimental.pallas import tpu as pltpu
from jax.experimental.pallas import tpu_sc as plsc

_NUM_ITEMS = 1000000
_ROWS = _NUM_ITEMS + 1
_DIM = 32
_BATCH = 16384

_NC = 2
_NS = 16
_NW = _NC * _NS
_B_PER_W = _BATCH // _NW
_CHUNK = 128
_NCHUNK = _B_PER_W // _CHUNK

_mesh = plsc.VectorSubcoreMesh(core_axis_name="c", subcore_axis_name="s")


@functools.partial(
    pl.kernel,
    mesh=_mesh,
    out_type=jax.ShapeDtypeStruct((_DIM, _BATCH), jnp.float32),
    scratch_types=[
        pltpu.VMEM((_B_PER_W,), jnp.int32),
        pltpu.VMEM((_DIM, _B_PER_W), jnp.int32),
        pltpu.VMEM((_DIM, _B_PER_W), jnp.float32),
        pltpu.SemaphoreType.DMA,
    ],
)
def _gather_kernel(idx_hbm, table_hbm, out_hbm, idx_v, widx_v, block_v, sem):
    wid = lax.axis_index("s") * _NC + lax.axis_index("c")
    base = wid * _B_PER_W

    pltpu.sync_copy(idx_hbm.at[pl.ds(base, _B_PER_W)], idx_v)

    @pl.loop(0, _B_PER_W // 16)
    def build(v):
        iv = idx_v[pl.ds(v * 16, 16)]
        for f in range(_DIM):
            widx_v[f, pl.ds(v * 16, 16)] = iv + f * _ROWS

    @pl.loop(0, _NCHUNK * _DIM // 8)
    def body(t):
        c = t // (_DIM // 8)
        f0 = (t % (_DIM // 8)) * 8
        copies = []
        for j in range(8):
            copies.append(
                pltpu.async_copy(
                    table_hbm.at[widx_v.at[f0 + j, pl.ds(c * _CHUNK, _CHUNK)]],
                    block_v.at[f0 + j, pl.ds(c * _CHUNK, _CHUNK)],
                    sem,
                )
            )
        for cp in copies:
            cp.wait()

    pltpu.sync_copy(block_v, out_hbm.at[:, pl.ds(base, _B_PER_W)])


def kernel(indices, emb_weight):
    table_f = emb_weight.T.reshape(_DIM * _ROWS)
    out_t = _gather_kernel(indices.astype(jnp.int32), table_f)
    return out_t.T


# speedup vs baseline: 7.1280x; 7.1280x over previous
import functools

import jax
import jax.numpy as jnp
from jax import lax
from jax.experimental import pallas as pl
from jax.experimental.pallas import tpu as pltpu
from jax.experimental.pallas import tpu_sc as plsc

_NUM_ITEMS = 1000000
_DIM = 32
_BATCH = 16384

_NC = 2
_NS = 16
_NW = _NC * _NS
_B_PER_W = _BATCH // _NW
_BB = 32
_NBATCH = _B_PER_W // _BB

_mesh = plsc.VectorSubcoreMesh(core_axis_name="c", subcore_axis_name="s")


@functools.partial(
    pl.kernel,
    mesh=_mesh,
    out_type=jax.ShapeDtypeStruct((_BATCH, _DIM), jnp.float32),
    scratch_types=[
        pltpu.VMEM((_B_PER_W,), jnp.int32),
        pltpu.VMEM((_BB, 8, _DIM), jnp.float32),
        pltpu.VMEM((_BB, _DIM), jnp.float32),
        pltpu.SemaphoreType.DMA,
    ],
)
def _gather_kernel(idx_hbm, table_hbm, out_hbm, idx_v, slab_v, stage_v, sem):
    wid = lax.axis_index("s") * _NC + lax.axis_index("c")
    base = wid * _B_PER_W

    pltpu.sync_copy(idx_hbm.at[pl.ds(base, _B_PER_W)], idx_v)

    @pl.loop(0, _NBATCH)
    def batch(b):
        i0 = b * _BB
        for h in range(2):
            iv = idx_v[pl.ds(i0 + h * 16, 16)]
            copies = []
            for j in range(16):
                k = h * 16 + j
                idx = iv[j]
                slab = pl.multiple_of((idx // 8) * 8, 8)
                copies.append(
                    pltpu.async_copy(
                        table_hbm.at[pl.ds(slab, 8), :],
                        slab_v.at[k],
                        sem,
                    )
                )
            for cp in copies:
                cp.wait()
        iv0 = idx_v[pl.ds(i0, 16)]
        iv1 = idx_v[pl.ds(i0 + 16, 16)]
        for k in range(_BB):
            r = (iv0[k] if k < 16 else iv1[k - 16]) % 8
            stage_v[k, pl.ds(0, 16)] = slab_v[k, r, pl.ds(0, 16)]
            stage_v[k, pl.ds(16, 16)] = slab_v[k, r, pl.ds(16, 16)]
        pltpu.sync_copy(stage_v, out_hbm.at[pl.ds(base + i0, _BB), :])


def kernel(indices, emb_weight):
    return _gather_kernel(indices.astype(jnp.int32), emb_weight)


# speedup vs baseline: 7.6031x; 1.0667x over previous
import functools

import jax
import jax.numpy as jnp
from jax import lax
from jax.experimental import pallas as pl
from jax.experimental.pallas import tpu as pltpu
from jax.experimental.pallas import tpu_sc as plsc

_NUM_ITEMS = 1000000
_DIM = 32
_BATCH = 16384

_NC = 2
_NS = 16
_NW = _NC * _NS
_B_PER_W = _BATCH // _NW
_BB = 32
_NBATCH = _B_PER_W // _BB

_mesh = plsc.VectorSubcoreMesh(core_axis_name="c", subcore_axis_name="s")


@functools.partial(
    pl.kernel,
    mesh=_mesh,
    out_type=jax.ShapeDtypeStruct((_BATCH, _DIM), jnp.float32),
    scratch_types=[
        pltpu.VMEM((_B_PER_W,), jnp.int32),
        pltpu.VMEM((_BB, 8, _DIM), jnp.float32),
        pltpu.VMEM((_BB, 8, _DIM), jnp.float32),
        pltpu.VMEM((_BB, _DIM), jnp.float32),
        pltpu.SemaphoreType.DMA,
        pltpu.SemaphoreType.DMA,
    ],
)
def _gather_kernel(
    idx_hbm, table_hbm, out_hbm, idx_v, slab_a, slab_b, stage_v, sem_a, sem_b
):
    wid = lax.axis_index("s") * _NC + lax.axis_index("c")
    base = wid * _B_PER_W

    pltpu.sync_copy(idx_hbm.at[pl.ds(base, _B_PER_W)], idx_v)

    def fire(b, slab, sem):
        i0 = b * _BB
        for h in range(2):
            iv = idx_v[pl.ds(i0 + h * 16, 16)]
            for j in range(16):
                k = h * 16 + j
                slab_row = pl.multiple_of((iv[j] // 8) * 8, 8)
                pltpu.async_copy(
                    table_hbm.at[pl.ds(slab_row, 8), :],
                    slab.at[k],
                    sem,
                )

    def finish(b, slab, sem):
        for k in range(_BB):
            pltpu.make_async_copy(
                table_hbm.at[pl.ds(0, 8), :], slab.at[k], sem
            ).wait()
        i0 = b * _BB
        iv0 = idx_v[pl.ds(i0, 16)]
        iv1 = idx_v[pl.ds(i0 + 16, 16)]
        for k in range(_BB):
            r = (iv0[k] if k < 16 else iv1[k - 16]) % 8
            stage_v[k, pl.ds(0, 16)] = slab[k, r, pl.ds(0, 16)]
            stage_v[k, pl.ds(16, 16)] = slab[k, r, pl.ds(16, 16)]
        pltpu.sync_copy(stage_v, out_hbm.at[pl.ds(base + i0, _BB), :])

    fire(0, slab_a, sem_a)
    fire(1, slab_b, sem_b)

    @pl.loop(0, _NBATCH - 2, step=2)
    def step(b):
        finish(b, slab_a, sem_a)
        fire(b + 2, slab_a, sem_a)
        finish(b + 1, slab_b, sem_b)
        fire(b + 3, slab_b, sem_b)

    finish(_NBATCH - 2, slab_a, sem_a)
    finish(_NBATCH - 1, slab_b, sem_b)


def kernel(indices, emb_weight):
    return _gather_kernel(indices.astype(jnp.int32), emb_weight)
